# baseline (speedup 1.0000x reference)
import functools

import jax
import jax.numpy as jnp
from jax.experimental import pallas as pl
from jax.experimental.pallas import tpu as pltpu

D_MODEL = 4096
D_HIDDEN = 2048
N_EXPERTS = 64
Z_COEF = 0.001
B_COEF = 0.01

BT = 512
CH = 512


def _router_kernel(x_ref, w1_ref, b1_ref, w2_ref, depth_ref, aux_ref,
                   psum_acc, csum_acc, lse2_acc,
                   *, n_tok_blocks, n_tokens):
    t = pl.program_id(0)

    xb = x_ref[...].astype(jnp.bfloat16)
    logits = None
    for c in range(0, D_HIDDEN, CH):
        hc = jnp.dot(xb, w1_ref[:, c:c + CH],
                     preferred_element_type=jnp.float32)
        hc = hc + b1_ref[:, c:c + CH]
        hc = hc * jax.nn.sigmoid(hc)
        lc = jnp.dot(hc, w2_ref[c:c + CH, :],
                     preferred_element_type=jnp.float32,
                     precision=jax.lax.Precision.DEFAULT)
        logits = lc if logits is None else logits + lc

    m = jnp.max(logits, axis=-1, keepdims=True)
    e = jnp.exp(logits - m)
    s = jnp.sum(e, axis=-1, keepdims=True)
    probs = e / s
    lse = m + jnp.log(s)

    iota = jax.lax.broadcasted_iota(jnp.int32, logits.shape, 1)
    idx = jnp.min(jnp.where(logits == m, iota, N_EXPERTS),
                  axis=-1, keepdims=True)
    depth_ref[...] = idx + 1

    onehot = (iota == idx).astype(jnp.float32)
    psum = jnp.sum(probs, axis=0, keepdims=True)
    csum = jnp.sum(onehot, axis=0, keepdims=True)
    l2 = jnp.sum(lse * lse, axis=0, keepdims=True)

    @pl.when(t == 0)
    def _():
        psum_acc[...] = psum
        csum_acc[...] = csum
        lse2_acc[...] = l2

    @pl.when(t > 0)
    def _():
        psum_acc[...] = psum_acc[...] + psum
        csum_acc[...] = csum_acc[...] + csum
        lse2_acc[...] = lse2_acc[...] + l2

    @pl.when(t == n_tok_blocks - 1)
    def _():
        z_loss = lse2_acc[...] / n_tokens
        bal = jnp.sum(csum_acc[...] * psum_acc[...],
                      axis=-1, keepdims=True)
        bal = bal * (N_EXPERTS / (n_tokens * float(n_tokens)))
        aux_ref[...] = Z_COEF * z_loss + B_COEF * bal


def _run(x_flat, W1, b1_2d, W2, *, interpret=False):
    n_tokens = x_flat.shape[0]
    n_tok_blocks = n_tokens // BT

    kern = functools.partial(
        _router_kernel,
        n_tok_blocks=n_tok_blocks,
        n_tokens=n_tokens,
    )
    depths, aux = pl.pallas_call(
        kern,
        grid=(n_tok_blocks,),
        in_specs=[
            pl.BlockSpec((BT, D_MODEL), lambda t: (t, 0)),
            pl.BlockSpec((D_MODEL, D_HIDDEN), lambda t: (0, 0)),
            pl.BlockSpec((1, D_HIDDEN), lambda t: (0, 0)),
            pl.BlockSpec((D_HIDDEN, N_EXPERTS), lambda t: (0, 0)),
        ],
        out_specs=[
            pl.BlockSpec((BT, 1), lambda t: (t, 0)),
            pl.BlockSpec((1, 1), lambda t: (0, 0)),
        ],
        out_shape=[
            jax.ShapeDtypeStruct((n_tokens, 1), jnp.int32),
            jax.ShapeDtypeStruct((1, 1), jnp.float32),
        ],
        scratch_shapes=[
            pltpu.VMEM((1, N_EXPERTS), jnp.float32),
            pltpu.VMEM((1, N_EXPERTS), jnp.float32),
            pltpu.VMEM((1, 1), jnp.float32),
        ],
        interpret=interpret,
    )(x_flat, W1.astype(jnp.bfloat16), b1_2d, W2)
    return depths, aux


def kernel(x, W1, b1, W2):
    batch, seq, d = x.shape
    x_flat = x.reshape(-1, d)
    b1_2d = b1.reshape(1, -1)
    depths, aux = _run(x_flat, W1, b1_2d, W2)
    return depths.reshape(batch, seq), aux[0, 0]


# speedup vs baseline: 1.0323x; 1.0323x over previous
import functools

import jax
import jax.numpy as jnp
from jax.experimental import pallas as pl
from jax.experimental.pallas import tpu as pltpu

D_MODEL = 4096
D_HIDDEN = 2048
N_EXPERTS = 64
Z_COEF = 0.001
B_COEF = 0.01

BT = 512


def _router_kernel(x_ref, w1_ref, b1_ref, w2_ref, depth_ref, aux_ref,
                   h_scr, psum_acc, csum_acc, lse2_acc,
                   *, n_tok_blocks, n_tokens):
    t = pl.program_id(0)
    cur = jax.lax.rem(t, 2)
    prv = jax.lax.rem(t + 1, 2)

    @pl.when(t < n_tok_blocks)
    def _():
        xb = x_ref[...].astype(jnp.bfloat16)
        h_scr[cur] = jnp.dot(xb, w1_ref[...],
                             preferred_element_type=jnp.float32)

    @pl.when(t > 0)
    def _():
        h = h_scr[prv] + b1_ref[...]
        h = h * jax.nn.sigmoid(h)
        logits = jnp.dot(h, w2_ref[...], preferred_element_type=jnp.float32,
                         precision=jax.lax.Precision.DEFAULT)

        m = jnp.max(logits, axis=-1, keepdims=True)
        e = jnp.exp(logits - m)
        s = jnp.sum(e, axis=-1, keepdims=True)
        probs = e / s
        lse = m + jnp.log(s)

        iota = jax.lax.broadcasted_iota(jnp.int32, logits.shape, 1)
        idx = jnp.min(jnp.where(logits == m, iota, N_EXPERTS),
                      axis=-1, keepdims=True)
        depth_ref[...] = idx + 1

        onehot = (iota == idx).astype(jnp.float32)
        psum = jnp.sum(probs, axis=0, keepdims=True)
        csum = jnp.sum(onehot, axis=0, keepdims=True)
        l2 = jnp.sum(lse * lse, axis=0, keepdims=True)

        @pl.when(t == 1)
        def _():
            psum_acc[...] = psum
            csum_acc[...] = csum
            lse2_acc[...] = l2

        @pl.when(t > 1)
        def _():
            psum_acc[...] = psum_acc[...] + psum
            csum_acc[...] = csum_acc[...] + csum
            lse2_acc[...] = lse2_acc[...] + l2

        @pl.when(t == n_tok_blocks)
        def _():
            z_loss = lse2_acc[...] / n_tokens
            bal = jnp.sum(csum_acc[...] * psum_acc[...],
                          axis=-1, keepdims=True)
            bal = bal * (N_EXPERTS / (n_tokens * float(n_tokens)))
            aux_ref[...] = Z_COEF * z_loss + B_COEF * bal


def _run(x_flat, W1, b1_2d, W2, *, interpret=False):
    n_tokens = x_flat.shape[0]
    n_tok_blocks = n_tokens // BT
    last = n_tok_blocks - 1

    kern = functools.partial(
        _router_kernel,
        n_tok_blocks=n_tok_blocks,
        n_tokens=n_tokens,
    )
    depths, aux = pl.pallas_call(
        kern,
        grid=(n_tok_blocks + 1,),
        in_specs=[
            pl.BlockSpec((BT, D_MODEL), lambda t: (jnp.minimum(t, last), 0)),
            pl.BlockSpec((D_MODEL, D_HIDDEN), lambda t: (0, 0)),
            pl.BlockSpec((1, D_HIDDEN), lambda t: (0, 0)),
            pl.BlockSpec((D_HIDDEN, N_EXPERTS), lambda t: (0, 0)),
        ],
        out_specs=[
            pl.BlockSpec((BT, 1), lambda t: (jnp.maximum(t - 1, 0), 0)),
            pl.BlockSpec((1, 1), lambda t: (0, 0)),
        ],
        out_shape=[
            jax.ShapeDtypeStruct((n_tokens, 1), jnp.int32),
            jax.ShapeDtypeStruct((1, 1), jnp.float32),
        ],
        scratch_shapes=[
            pltpu.VMEM((2, BT, D_HIDDEN), jnp.float32),
            pltpu.VMEM((1, N_EXPERTS), jnp.float32),
            pltpu.VMEM((1, N_EXPERTS), jnp.float32),
            pltpu.VMEM((1, 1), jnp.float32),
        ],
        interpret=interpret,
    )(x_flat, W1.astype(jnp.bfloat16), b1_2d, W2)
    return depths, aux


def kernel(x, W1, b1, W2):
    batch, seq, d = x.shape
    x_flat = x.reshape(-1, d)
    b1_2d = b1.reshape(1, -1)
    depths, aux = _run(x_flat, W1, b1_2d, W2)
    return depths.reshape(batch, seq), aux[0, 0]


# speedup vs baseline: 1.0510x; 1.0181x over previous
import functools

import jax
import jax.numpy as jnp
from jax.experimental import pallas as pl
from jax.experimental.pallas import tpu as pltpu

D_MODEL = 4096
D_HIDDEN = 2048
N_EXPERTS = 64
Z_COEF = 0.001
B_COEF = 0.01

BT = 512


def _router_kernel(x_ref, w1_ref, b1_ref, w2_ref, depth_ref, aux_ref,
                   psum_acc, csum_acc, lse2_acc,
                   *, n_tok_blocks, n_tokens):
    t = pl.program_id(0)

    xb = x_ref[...].astype(jnp.bfloat16)
    CM = 256
    parts = []
    for r in range(0, BT, CM):
        hr = jnp.dot(xb[r:r + CM], w1_ref[...],
                     preferred_element_type=jnp.float32)
        hr = hr + b1_ref[...]
        hr = hr * jax.nn.sigmoid(hr)
        parts.append(jnp.dot(hr, w2_ref[...],
                             preferred_element_type=jnp.float32,
                             precision=jax.lax.Precision.DEFAULT))
    logits = jnp.concatenate(parts, axis=0)

    m = jnp.max(logits, axis=-1, keepdims=True)
    e = jnp.exp(logits - m)
    s = jnp.sum(e, axis=-1, keepdims=True)
    probs = e / s
    lse = m + jnp.log(s)

    iota = jax.lax.broadcasted_iota(jnp.int32, logits.shape, 1)
    idx = jnp.min(jnp.where(logits == m, iota, N_EXPERTS),
                  axis=-1, keepdims=True)
    depth_ref[...] = idx + 1

    onehot = (iota == idx).astype(jnp.float32)
    psum = jnp.sum(probs, axis=0, keepdims=True)
    csum = jnp.sum(onehot, axis=0, keepdims=True)
    l2 = jnp.sum(lse * lse, axis=0, keepdims=True)

    @pl.when(t == 0)
    def _():
        psum_acc[...] = psum
        csum_acc[...] = csum
        lse2_acc[...] = l2

    @pl.when(t > 0)
    def _():
        psum_acc[...] = psum_acc[...] + psum
        csum_acc[...] = csum_acc[...] + csum
        lse2_acc[...] = lse2_acc[...] + l2

    @pl.when(t == n_tok_blocks - 1)
    def _():
        z_loss = lse2_acc[...] / n_tokens
        bal = jnp.sum(csum_acc[...] * psum_acc[...],
                      axis=-1, keepdims=True)
        bal = bal * (N_EXPERTS / (n_tokens * float(n_tokens)))
        aux_ref[...] = Z_COEF * z_loss + B_COEF * bal


def _run(x_flat, W1, b1_2d, W2, *, interpret=False):
    n_tokens = x_flat.shape[0]
    n_tok_blocks = n_tokens // BT

    kern = functools.partial(
        _router_kernel,
        n_tok_blocks=n_tok_blocks,
        n_tokens=n_tokens,
    )
    depths, aux = pl.pallas_call(
        kern,
        grid=(n_tok_blocks,),
        in_specs=[
            pl.BlockSpec((BT, D_MODEL), lambda t: (t, 0)),
            pl.BlockSpec((D_MODEL, D_HIDDEN), lambda t: (0, 0)),
            pl.BlockSpec((1, D_HIDDEN), lambda t: (0, 0)),
            pl.BlockSpec((D_HIDDEN, N_EXPERTS), lambda t: (0, 0)),
        ],
        out_specs=[
            pl.BlockSpec((BT, 1), lambda t: (t, 0)),
            pl.BlockSpec((1, 1), lambda t: (0, 0)),
        ],
        out_shape=[
            jax.ShapeDtypeStruct((n_tokens, 1), jnp.int32),
            jax.ShapeDtypeStruct((1, 1), jnp.float32),
        ],
        scratch_shapes=[
            pltpu.VMEM((1, N_EXPERTS), jnp.float32),
            pltpu.VMEM((1, N_EXPERTS), jnp.float32),
            pltpu.VMEM((1, 1), jnp.float32),
        ],
        interpret=interpret,
    )(x_flat, W1.astype(jnp.bfloat16), b1_2d, W2)
    return depths, aux


def kernel(x, W1, b1, W2):
    batch, seq, d = x.shape
    x_flat = x.reshape(-1, d)
    b1_2d = b1.reshape(1, -1)
    depths, aux = _run(x_flat, W1, b1_2d, W2)
    return depths.reshape(batch, seq), aux[0, 0]


# speedup vs baseline: 1.0825x; 1.0299x over previous
import functools

import jax
import jax.numpy as jnp
from jax.experimental import pallas as pl
from jax.experimental.pallas import tpu as pltpu

D_MODEL = 4096
D_HIDDEN = 2048
N_EXPERTS = 64
Z_COEF = 0.001
B_COEF = 0.01

BT = 512


def _router_kernel(x_ref, w1_ref, b1_ref, w2_ref, depth_ref, aux_ref,
                   psum_acc, csum_acc, lse2_acc,
                   *, n_tok_blocks, n_tokens):
    t = pl.program_id(0)

    xb = x_ref[...].astype(jnp.bfloat16)
    h = jnp.dot(xb, w1_ref[...], preferred_element_type=jnp.float32)
    h = h + b1_ref[...]
    h = h * jax.nn.sigmoid(h)
    logits = jnp.dot(h, w2_ref[...], preferred_element_type=jnp.float32,
                     precision=jax.lax.Precision.DEFAULT)

    m = jnp.max(logits, axis=-1, keepdims=True)
    e = jnp.exp(logits - m)
    s = jnp.sum(e, axis=-1, keepdims=True)
    probs = e / s
    lse = m + jnp.log(s)

    iota = jax.lax.broadcasted_iota(jnp.int32, logits.shape, 1)
    idx = jnp.min(jnp.where(logits == m, iota, N_EXPERTS),
                  axis=-1, keepdims=True)
    depth_ref[...] = idx + 1

    onehot = (iota == idx).astype(jnp.float32)
    psum = jnp.sum(probs, axis=0, keepdims=True)
    csum = jnp.sum(onehot, axis=0, keepdims=True)
    l2 = jnp.sum(lse * lse, axis=0, keepdims=True)

    @pl.when(t == 0)
    def _():
        psum_acc[...] = psum
        csum_acc[...] = csum
        lse2_acc[...] = l2

    @pl.when(t > 0)
    def _():
        psum_acc[...] = psum_acc[...] + psum
        csum_acc[...] = csum_acc[...] + csum
        lse2_acc[...] = lse2_acc[...] + l2

    @pl.when(t == n_tok_blocks - 1)
    def _():
        z_loss = lse2_acc[...] / n_tokens
        bal = jnp.sum(csum_acc[...] * psum_acc[...],
                      axis=-1, keepdims=True)
        bal = bal * (N_EXPERTS / (n_tokens * float(n_tokens)))
        aux_ref[...] = Z_COEF * z_loss + B_COEF * bal


def _run(x_flat, W1, b1_2d, W2, *, interpret=False):
    n_tokens = x_flat.shape[0]
    n_tok_blocks = n_tokens // BT

    kern = functools.partial(
        _router_kernel,
        n_tok_blocks=n_tok_blocks,
        n_tokens=n_tokens,
    )
    depths, aux = pl.pallas_call(
        kern,
        grid=(n_tok_blocks,),
        in_specs=[
            pl.BlockSpec((BT, D_MODEL), lambda t: (t, 0)),
            pl.BlockSpec((D_MODEL, D_HIDDEN), lambda t: (0, 0)),
            pl.BlockSpec((1, D_HIDDEN), lambda t: (0, 0)),
            pl.BlockSpec((D_HIDDEN, N_EXPERTS), lambda t: (0, 0)),
        ],
        out_specs=[
            pl.BlockSpec((BT, 1), lambda t: (t, 0)),
            pl.BlockSpec((1, 1), lambda t: (0, 0)),
        ],
        out_shape=[
            jax.ShapeDtypeStruct((n_tokens, 1), jnp.int32),
            jax.ShapeDtypeStruct((1, 1), jnp.float32),
        ],
        scratch_shapes=[
            pltpu.VMEM((1, N_EXPERTS), jnp.float32),
            pltpu.VMEM((1, N_EXPERTS), jnp.float32),
            pltpu.VMEM((1, 1), jnp.float32),
        ],
        interpret=interpret,
    )(x_flat, W1.astype(jnp.bfloat16), b1_2d, W2)
    return depths, aux


def kernel(x, W1, b1, W2):
    batch, seq, d = x.shape
    x_flat = x.reshape(-1, d)
    b1_2d = b1.reshape(1, -1)
    depths, aux = _run(x_flat, W1, b1_2d, W2)
    return depths.reshape(batch, seq), aux[0, 0]
